# baseline (speedup 1.0000x reference)
import functools

import jax
import jax.numpy as jnp
from jax.experimental import pallas as pl
from jax.experimental.pallas import tpu as pltpu
from jax.experimental.pallas import tpu_sc as plsc


def _tc_body(vis_ref, tac_ref, wv_ref, wt_ref, bp_ref, x_ref, cx_ref, cy_ref):
    cv = vis_ref.shape[2]
    v = vis_ref[0]
    t = tac_ref[0]
    x_ref[0, :, 0:cv] = v
    x_ref[0, :, cv:] = t
    dims = (((0,), (1,)), ((), ()))
    cT = (jax.lax.dot_general(wv_ref[...], v, dims,
                              preferred_element_type=jnp.float32)
          + jax.lax.dot_general(wt_ref[...], t, dims,
                                preferred_element_type=jnp.float32)
          + bp_ref[...])
    cx_ref[0] = cT[0:1, :]
    cy_ref[0] = cT[1:2, :]


def _sc_body(E, N, EH, EHP, cxa_hbm, cya_hbm, srcp_hbm, dstp_hbm, eif_hbm,
             attr_hbm, eibf_hbm, cx_v, cy_v, si_v, di_v, av_v, ei_v, eo_v):
    B = cxa_hbm.shape[0]
    c = jax.lax.axis_index("c")
    s = jax.lax.axis_index("s")
    wid = s * 2 + c
    b = wid % B
    h = wid // B

    pltpu.sync_copy(cxa_hbm.at[b, 0], cx_v)
    pltpu.sync_copy(cya_hbm.at[b, 0], cy_v)
    pltpu.sync_copy(srcp_hbm.at[pl.ds(h * EH, EHP)], si_v)
    pltpu.sync_copy(dstp_hbm.at[pl.ds(h * EH, EHP)], di_v)

    def edge_step(i, carry):
        idx_s = si_v[pl.ds(i * 16, 16)]
        idx_d = di_v[pl.ds(i * 16, 16)]
        xs = plsc.load_gather(cx_v, [idx_s])
        xd = plsc.load_gather(cx_v, [idx_d])
        ys = plsc.load_gather(cy_v, [idx_s])
        yd = plsc.load_gather(cy_v, [idx_d])
        dx = xs - xd
        dy = ys - yd
        s2 = dx * dx + dy * dy
        s2 = jnp.where(s2 < 1e-37, 0.0, s2)
        u = jnp.zeros((16,), jnp.float32) + 1.0
        for t in (63, 32, 16, 8, 4, 2, 1, 1):
            x_t = (s2 * u) * u
            big = x_t >= (2.0 ** (2 * t))
            small = x_t < (2.0 ** (-2 * t))
            u = jnp.where(big, u * (2.0 ** (-t)),
                          jnp.where(small, u * (2.0 ** t), u))
        x_r = (s2 * u) * u
        yr = 1.437 - 0.28 * x_r
        for _ in range(5):
            yr = yr * (1.5 - 0.5 * x_r * yr * yr)
        dist = (s2 * yr) * u
        w = 1.0 / (dist + 1e-6)
        av_v[pl.ds(i * 16, 16)] = 1.0 / (1.0 + jnp.exp(-w))
        return carry

    jax.lax.fori_loop(0, EHP // 16, edge_step, 0)
    pltpu.sync_copy(av_v.at[pl.ds(0, EH)],
                    attr_hbm.at[pl.ds(b * E + h * EH, EH)])

    pltpu.sync_copy(eif_hbm.at[pl.ds(h * E, E)], ei_v)

    def eib_step(i, carry):
        eo_v[pl.ds(i * 16, 16)] = ei_v[pl.ds(i * 16, 16)] + b * N
        return carry

    jax.lax.fori_loop(0, E // 16, eib_step, 0)
    pltpu.sync_copy(eo_v, eibf_hbm.at[pl.ds(h * B * E + b * E, E)])


def kernel(visual_feat, tactile_feat, Wp, bp, edge_index):
    B, Cv, H, W = visual_feat.shape
    Ct = tactile_feat.shape[1]
    C = Cv + Ct
    N = H * W
    E = edge_index.shape[1]
    EH = E // 2
    EHP = (EH // 16 + 1) * 16

    vis = jnp.transpose(visual_feat, (0, 2, 3, 1)).reshape(B, N, Cv)
    tac = jnp.transpose(tactile_feat, (0, 2, 3, 1)).reshape(B, N, Ct)
    wv = Wp[:Cv]
    wt = Wp[Cv:]
    bp2 = bp.reshape(2, 1)
    ei = edge_index.astype(jnp.int32)
    pad = 2 * EHP - E
    srcp = jnp.pad(ei[0], (0, pad))
    dstp = jnp.pad(ei[1], (0, pad))
    eif = ei.reshape(2 * E)

    x_out, cxa, cya = pl.pallas_call(
        _tc_body,
        grid=(B,),
        in_specs=[
            pl.BlockSpec((1, N, Cv), lambda b: (b, 0, 0)),
            pl.BlockSpec((1, N, Ct), lambda b: (b, 0, 0)),
            pl.BlockSpec((Cv, 2), lambda b: (0, 0)),
            pl.BlockSpec((Ct, 2), lambda b: (0, 0)),
            pl.BlockSpec((2, 1), lambda b: (0, 0)),
        ],
        out_specs=[
            pl.BlockSpec((1, N, C), lambda b: (b, 0, 0)),
            pl.BlockSpec((1, 1, N), lambda b: (b, 0, 0)),
            pl.BlockSpec((1, 1, N), lambda b: (b, 0, 0)),
        ],
        out_shape=[
            jax.ShapeDtypeStruct((B, N, C), jnp.float32),
            jax.ShapeDtypeStruct((B, 1, N), jnp.float32),
            jax.ShapeDtypeStruct((B, 1, N), jnp.float32),
        ],
    )(vis, tac, wv, wt, bp2)

    sc_fn = pl.kernel(
        functools.partial(_sc_body, E, N, EH, EHP),
        out_type=[
            jax.ShapeDtypeStruct((B * E,), jnp.float32),
            jax.ShapeDtypeStruct((2 * B * E,), jnp.int32),
        ],
        mesh=plsc.VectorSubcoreMesh(core_axis_name="c", subcore_axis_name="s"),
        compiler_params=pltpu.CompilerParams(needs_layout_passes=False),
        scratch_types=[
            pltpu.VMEM((N,), jnp.float32),
            pltpu.VMEM((N,), jnp.float32),
            pltpu.VMEM((EHP,), jnp.int32),
            pltpu.VMEM((EHP,), jnp.int32),
            pltpu.VMEM((EHP,), jnp.float32),
            pltpu.VMEM((E,), jnp.int32),
            pltpu.VMEM((E,), jnp.int32),
        ],
    )
    attr_flat, eibf = sc_fn(cxa, cya, srcp, dstp, eif)

    x_batched = x_out.reshape(B * N, C)
    edge_index_batched = eibf.reshape(2, B * E).astype(edge_index.dtype)
    edge_attr_batched = attr_flat.reshape(B * E, 1)
    return (x_batched, edge_index_batched, edge_attr_batched)
